# baseline (speedup 1.0000x reference)
import functools

import numpy as np
import jax
import jax.numpy as jnp
from jax.experimental import pallas as pl
from jax.experimental.pallas import tpu as pltpu


@functools.lru_cache(maxsize=None)
def _pos_encoding(seq_len: int, d: int):
    pos = np.arange(seq_len, dtype=np.float32)[:, None]
    i = np.arange(0, d, 2, dtype=np.float32)[None, :]
    angle = pos / np.power(10000.0, i / d)
    pe = np.zeros((seq_len, d), dtype=np.float32)
    pe[:, 0::2] = np.sin(angle)
    pe[:, 1::2] = np.cos(angle)
    return jnp.asarray(pe)


def _gcn_body(adj_ref, ids_ref, w_ref, b_ref, pe_ref, out_ref, h_ref):
    n_s = pe_ref.shape[0]
    n_q = ids_ref.shape[0]
    embed = w_ref.shape[1]

    @pl.when(pl.program_id(0) == 0)
    def _():
        h_ref[...] = jnp.dot(ids_ref[...], w_ref[...],
                             preferred_element_type=jnp.float32)

    h = h_ref[...]
    a = adj_ref[...].astype(jnp.bfloat16)
    deg = jnp.sum(a, axis=1).astype(jnp.float32)
    dinv = jnp.where(deg > 0.0, jax.lax.rsqrt(jnp.maximum(deg, 1.0)), 0.0)
    dinv = dinv.astype(jnp.bfloat16)
    a_t = jnp.transpose(a, (0, 2, 1))
    c = a_t * dinv[:, :, None] * dinv[:, None, :]
    out = jnp.dot(c.reshape(n_s * n_q, n_q), h.astype(jnp.bfloat16),
                  preferred_element_type=jnp.float32)
    out = out + b_ref[...]
    out = out.reshape(n_s, n_q, embed) + pe_ref[...][:, None, :]
    out_ref[...] = out.reshape(1, n_s, n_q, embed)


def kernel(adj_matrices, ids, W, b):
    bsz, n_slices, n_q, _ = adj_matrices.shape
    embed = W.shape[1]
    pe = _pos_encoding(n_slices, embed)
    b2 = b.reshape(1, embed)

    grid = (bsz,)
    return pl.pallas_call(
        _gcn_body,
        grid=grid,
        in_specs=[
            pl.BlockSpec((n_slices, n_q, n_q), lambda g: (g, 0, 0)),
            pl.BlockSpec((n_q, embed), lambda g: (0, 0)),
            pl.BlockSpec((embed, embed), lambda g: (0, 0)),
            pl.BlockSpec((1, embed), lambda g: (0, 0)),
            pl.BlockSpec((n_slices, embed), lambda g: (0, 0)),
        ],
        out_specs=pl.BlockSpec((1, n_slices, n_q, embed),
                               lambda g: (g, 0, 0, 0)),
        out_shape=jax.ShapeDtypeStruct((bsz, n_slices, n_q, embed),
                                       jnp.float32),
        scratch_shapes=[pltpu.VMEM((n_q, embed), jnp.float32)],
        compiler_params=pltpu.CompilerParams(
            dimension_semantics=("arbitrary",),
        ),
    )(adj_matrices.reshape(bsz * n_slices, n_q, n_q), ids[:n_q], W, b2, pe)


# speedup vs baseline: 1.5982x; 1.5982x over previous
import functools

import numpy as np
import jax
import jax.numpy as jnp
from jax.experimental import pallas as pl
from jax.experimental.pallas import tpu as pltpu


@functools.lru_cache(maxsize=None)
def _pos_encoding(seq_len: int, d: int):
    pos = np.arange(seq_len, dtype=np.float32)[:, None]
    i = np.arange(0, d, 2, dtype=np.float32)[None, :]
    angle = pos / np.power(10000.0, i / d)
    pe = np.zeros((seq_len, d), dtype=np.float32)
    pe[:, 0::2] = np.sin(angle)
    pe[:, 1::2] = np.cos(angle)
    return jnp.asarray(pe)


def _gcn_body(adj_ref, ids_ref, w_ref, b_ref, pe_ref, k_ref, out_ref, h_ref):
    n_s = pe_ref.shape[0]
    n_q = ids_ref.shape[0]
    embed = w_ref.shape[1]

    @pl.when(pl.program_id(0) == 0)
    def _():
        h_ref[...] = jnp.dot(ids_ref[...], w_ref[...],
                             preferred_element_type=jnp.float32)

    h = h_ref[...]
    a2 = adj_ref[0].astype(jnp.bfloat16)
    deg = jnp.dot(a2, k_ref[...], preferred_element_type=jnp.float32)
    a = a2.reshape(n_s, n_q, n_q)
    dinv = jnp.where(deg > 0.0, jax.lax.rsqrt(jnp.maximum(deg, 1.0)), 0.0)
    dinv = dinv.astype(jnp.bfloat16)
    a_t = jnp.transpose(a, (0, 2, 1))
    c = a_t * dinv[:, :, None] * dinv[:, None, :]
    out = jnp.dot(c.reshape(n_s * n_q, n_q), h.astype(jnp.bfloat16),
                  preferred_element_type=jnp.float32)
    out = out + b_ref[...]
    out = out.reshape(n_s, n_q, embed) + pe_ref[...][:, None, :]
    out_ref[...] = out.reshape(1, n_s, n_q, embed)


def kernel(adj_matrices, ids, W, b):
    bsz, n_slices, n_q, _ = adj_matrices.shape
    embed = W.shape[1]
    pe = _pos_encoding(n_slices, embed)
    b2 = b.reshape(1, embed)
    kmat = jnp.asarray(np.tile(np.eye(n_q, dtype=np.float32), (n_q, 1)),
                       dtype=jnp.bfloat16)

    grid = (bsz,)
    return pl.pallas_call(
        _gcn_body,
        grid=grid,
        in_specs=[
            pl.BlockSpec((1, n_slices, n_q * n_q), lambda g: (g, 0, 0)),
            pl.BlockSpec((n_q, embed), lambda g: (0, 0)),
            pl.BlockSpec((embed, embed), lambda g: (0, 0)),
            pl.BlockSpec((1, embed), lambda g: (0, 0)),
            pl.BlockSpec((n_slices, embed), lambda g: (0, 0)),
            pl.BlockSpec((n_q * n_q, n_q), lambda g: (0, 0)),
        ],
        out_specs=pl.BlockSpec((1, n_slices, n_q, embed),
                               lambda g: (g, 0, 0, 0)),
        out_shape=jax.ShapeDtypeStruct((bsz, n_slices, n_q, embed),
                                       jnp.float32),
        scratch_shapes=[pltpu.VMEM((n_q, embed), jnp.float32)],
        compiler_params=pltpu.CompilerParams(
            dimension_semantics=("arbitrary",),
        ),
    )(adj_matrices.reshape(bsz, n_slices, n_q * n_q), ids[:n_q], W, b2, pe,
      kmat)


# speedup vs baseline: 1.6013x; 1.0019x over previous
import functools

import numpy as np
import jax
import jax.numpy as jnp
from jax.experimental import pallas as pl
from jax.experimental.pallas import tpu as pltpu


@functools.lru_cache(maxsize=None)
def _pos_encoding(seq_len: int, d: int):
    pos = np.arange(seq_len, dtype=np.float32)[:, None]
    i = np.arange(0, d, 2, dtype=np.float32)[None, :]
    angle = pos / np.power(10000.0, i / d)
    pe = np.zeros((seq_len, d), dtype=np.float32)
    pe[:, 0::2] = np.sin(angle)
    pe[:, 1::2] = np.cos(angle)
    return jnp.asarray(pe)


def _gcn_body(adj_ref, ids_ref, w_ref, b_ref, pe_ref, k_ref, out_ref, h_ref):
    n_s = pe_ref.shape[0]
    n_q = ids_ref.shape[0]
    embed = w_ref.shape[1]

    @pl.when(pl.program_id(0) == 0)
    def _():
        h_ref[...] = jnp.dot(ids_ref[...], w_ref[...],
                             preferred_element_type=jnp.float32)

    h = h_ref[...]
    a2 = adj_ref[0].astype(jnp.bfloat16)
    deg = jnp.dot(a2, k_ref[...],
                  preferred_element_type=jnp.float32)[:, :n_q]
    a = a2.reshape(n_s, n_q, n_q)
    dinv = jnp.where(deg > 0.0, jax.lax.rsqrt(jnp.maximum(deg, 1.0)), 0.0)
    dinv = dinv.astype(jnp.bfloat16)
    a_t = jnp.transpose(a, (0, 2, 1))
    c = a_t * dinv[:, :, None] * dinv[:, None, :]
    out = jnp.dot(c.reshape(n_s * n_q, n_q), h.astype(jnp.bfloat16),
                  preferred_element_type=jnp.float32)
    out = out + b_ref[...]
    out = out.reshape(n_s, n_q, embed) + pe_ref[...][:, None, :]
    out_ref[...] = out.reshape(1, n_s, n_q, embed)


def kernel(adj_matrices, ids, W, b):
    bsz, n_slices, n_q, _ = adj_matrices.shape
    embed = W.shape[1]
    pe = _pos_encoding(n_slices, embed)
    b2 = b.reshape(1, embed)
    lanes = max(n_q, 128)
    kn = np.zeros((n_q * n_q, lanes), dtype=np.float32)
    kn[:, :n_q] = np.tile(np.eye(n_q, dtype=np.float32), (n_q, 1))
    kmat = jnp.asarray(kn, dtype=jnp.bfloat16)

    grid = (bsz,)
    return pl.pallas_call(
        _gcn_body,
        grid=grid,
        in_specs=[
            pl.BlockSpec((1, n_slices, n_q * n_q), lambda g: (g, 0, 0)),
            pl.BlockSpec((n_q, embed), lambda g: (0, 0)),
            pl.BlockSpec((embed, embed), lambda g: (0, 0)),
            pl.BlockSpec((1, embed), lambda g: (0, 0)),
            pl.BlockSpec((n_slices, embed), lambda g: (0, 0)),
            pl.BlockSpec((n_q * n_q, max(n_q, 128)), lambda g: (0, 0)),
        ],
        out_specs=pl.BlockSpec((1, n_slices, n_q, embed),
                               lambda g: (g, 0, 0, 0)),
        out_shape=jax.ShapeDtypeStruct((bsz, n_slices, n_q, embed),
                                       jnp.float32),
        scratch_shapes=[pltpu.VMEM((n_q, embed), jnp.float32)],
        compiler_params=pltpu.CompilerParams(
            dimension_semantics=("arbitrary",),
        ),
    )(adj_matrices.reshape(bsz, n_slices, n_q * n_q), ids[:n_q], W, b2, pe,
      kmat)
